# Initial kernel scaffold; baseline (speedup 1.0000x reference)
import jax
import jax.numpy as jnp
from jax.experimental import pallas as pl


def kernel(x, time_step, W1, b1, W2, b2, W3, b3, joint_pos, joint_vel, body_pos_w, body_quat_w, body_lin_vel_w, body_ang_vel_w):
    raise NotImplementedError("write your pallas kernel here")


# speedup vs baseline: 1.1365x; 1.1365x over previous
import functools

import jax
import jax.numpy as jnp
from jax import lax
from jax.experimental import pallas as pl
from jax.experimental.pallas import tpu as pltpu
from jax.experimental.pallas import tpu_sc as plsc

_T_TOTAL = 100000
_OBS = 512
_H1 = 256
_H2 = 128
_ACT = 32
_NJ = 32
_NB = 16
_BATCH = 4096

_NC, _NS, _L = 2, 16, 16
_NW = _NC * _NS
_BPW = _BATCH // _NW

_DIMS = (_NJ, _NJ, _NB * 3, _NB * 4, _NB * 3, _NB * 3)

_sc_mesh = plsc.VectorSubcoreMesh(
    core_axis_name="c", subcore_axis_name="s",
    num_cores=_NC, num_subcores=_NS)


def _gather_body(ts_hbm, t0, t1, t2, t3, t4, t5,
                 o0, o1, o2, o3, o4, o5,
                 idx_v, b0, b1, b2, b3, b4, b5, sem):
    wid = lax.axis_index("s") * _NC + lax.axis_index("c")
    base = wid * _BPW
    pltpu.sync_copy(ts_hbm.at[pl.ds(base, _BPW)], idx_v)
    for i in range(_BPW // _L):
        sl = pl.ds(i * _L, _L)
        idx_v[sl] = jnp.minimum(idx_v[sl], _T_TOTAL - 1)
    tabs = (t0, t1, t2, t3, t4, t5)
    bufs = (b0, b1, b2, b3, b4, b5)
    outs = (o0, o1, o2, o3, o4, o5)
    cps = [pltpu.async_copy(tab.at[idx_v], buf, sem)
           for tab, buf in zip(tabs, bufs)]
    for cp, buf, out in zip(cps, bufs, outs):
        cp.wait()
        pltpu.sync_copy(buf, out.at[pl.ds(base, _BPW)])


_gather_call = pl.kernel(
    _gather_body,
    out_type=tuple(jax.ShapeDtypeStruct((_BATCH, d), jnp.float32)
                   for d in _DIMS),
    mesh=_sc_mesh,
    scratch_types=[pltpu.VMEM((_BPW,), jnp.int32)]
                  + [pltpu.VMEM((_BPW, d), jnp.float32) for d in _DIMS]
                  + [pltpu.SemaphoreType.DMA],
    compiler_params=pltpu.CompilerParams(use_tc_tiling_on_sc=False),
)


_BM = 512


def _mlp_body(x_ref, w1_ref, b1_ref, w2_ref, b2_ref, w3_ref, b3_ref, o_ref):
    h = jnp.dot(x_ref[...], w1_ref[...],
                preferred_element_type=jnp.float32) + b1_ref[...]
    h = jnp.where(h > 0, h, jnp.exp(h) - 1.0)
    h = jnp.dot(h, w2_ref[...], preferred_element_type=jnp.float32) + b2_ref[...]
    h = jnp.where(h > 0, h, jnp.exp(h) - 1.0)
    o_ref[...] = jnp.dot(h, w3_ref[...],
                         preferred_element_type=jnp.float32) + b3_ref[...]


_mlp_call = pl.pallas_call(
    _mlp_body,
    grid=(_BATCH // _BM,),
    in_specs=[
        pl.BlockSpec((_BM, _OBS), lambda i: (i, 0)),
        pl.BlockSpec((_OBS, _H1), lambda i: (0, 0)),
        pl.BlockSpec((1, _H1), lambda i: (0, 0)),
        pl.BlockSpec((_H1, _H2), lambda i: (0, 0)),
        pl.BlockSpec((1, _H2), lambda i: (0, 0)),
        pl.BlockSpec((_H2, _ACT), lambda i: (0, 0)),
        pl.BlockSpec((1, _ACT), lambda i: (0, 0)),
    ],
    out_specs=pl.BlockSpec((_BM, _ACT), lambda i: (i, 0)),
    out_shape=jax.ShapeDtypeStruct((_BATCH, _ACT), jnp.float32),
)


def kernel(x, time_step, W1, b1, W2, b2, W3, b3,
           joint_pos, joint_vel, body_pos_w, body_quat_w,
           body_lin_vel_w, body_ang_vel_w):
    ts = time_step.astype(jnp.int32).reshape(_BATCH)
    g_jp, g_jv, g_bp, g_bq, g_blv, g_bav = _gather_call(
        ts,
        joint_pos,
        joint_vel,
        body_pos_w.reshape(_T_TOTAL, _NB * 3),
        body_quat_w.reshape(_T_TOTAL, _NB * 4),
        body_lin_vel_w.reshape(_T_TOTAL, _NB * 3),
        body_ang_vel_w.reshape(_T_TOTAL, _NB * 3),
    )
    actions = _mlp_call(x, W1, b1.reshape(1, _H1), W2, b2.reshape(1, _H2),
                        W3, b3.reshape(1, _ACT))
    return (actions,
            g_jp,
            g_jv,
            g_bp.reshape(_BATCH, _NB, 3),
            g_bq.reshape(_BATCH, _NB, 4),
            g_blv.reshape(_BATCH, _NB, 3),
            g_bav.reshape(_BATCH, _NB, 3))
